# baseline (device time: 764154 ns/iter reference)
import functools

import jax
import jax.numpy as jnp
from jax import lax
from jax.experimental import pallas as pl
from jax.experimental.pallas import tpu as pltpu

N_DEV = 16
COMM_DTYPE = jnp.bfloat16


def kernel(x, w_mat, scale_x, scale_w):
    m, k_local = x.shape
    _, n = w_mat.shape
    mb = m // N_DEV

    def body(x_ref, w_ref, sx_ref, sw_ref, out_ref,
             comm_ref, send_sems, recv_sems, credit_sem):
        my = lax.axis_index("i")
        left = lax.rem(my - 1 + N_DEV, N_DEV)
        right = lax.rem(my + 1, N_DEV)

        barrier_sem = pltpu.get_barrier_semaphore()
        for nbr in (left, right):
            pl.semaphore_signal(barrier_sem, inc=1, device_id=(nbr,),
                                device_id_type=pl.DeviceIdType.MESH)
        pl.semaphore_wait(barrier_sem, 2)

        w_bf16 = w_ref[...].astype(jnp.bfloat16)

        def contrib(o):
            xs = x_ref[pl.ds(o * mb, mb), :].astype(jnp.bfloat16)
            return lax.dot_general(
                xs, w_bf16, (((1,), (0,)), ((), ())),
                preferred_element_type=jnp.float32)

        o0 = lax.rem(my - 1 + N_DEV, N_DEV)
        comm_ref[1] = contrib(o0).astype(COMM_DTYPE)

        for s in range(N_DEV - 1):
            src_slot = (s - 1) % 2
            dst_slot = s % 2
            if s >= 2:
                pl.semaphore_wait(credit_sem, 1)
            rdma = pltpu.make_async_remote_copy(
                src_ref=comm_ref.at[src_slot],
                dst_ref=comm_ref.at[dst_slot],
                send_sem=send_sems.at[src_slot],
                recv_sem=recv_sems.at[dst_slot],
                device_id=(right,),
                device_id_type=pl.DeviceIdType.MESH,
            )
            rdma.start()
            rdma.wait()
            if 1 <= s <= 13:
                pl.semaphore_signal(credit_sem, inc=1, device_id=(left,),
                                    device_id_type=pl.DeviceIdType.MESH)
            o = lax.rem(my - 2 - s + 2 * N_DEV, N_DEV)
            merged = comm_ref[dst_slot].astype(jnp.float32) + contrib(o)
            if s < N_DEV - 2:
                comm_ref[dst_slot] = merged.astype(COMM_DTYPE)
            else:
                scale = sx_ref[0] * sw_ref[0]
                y = merged * scale
                out_ref[...] = y * jax.nn.sigmoid(y)

        @functools.partial(pl.run_scoped,
                           second_barrier=pltpu.SemaphoreType.REGULAR)
        def _(second_barrier):
            for nbr in (left, right):
                pl.semaphore_signal(second_barrier, inc=1, device_id=(nbr,),
                                    device_id_type=pl.DeviceIdType.MESH)
            pl.semaphore_wait(second_barrier, 2)

    return pl.pallas_call(
        body,
        out_shape=jax.ShapeDtypeStruct((mb, n), jnp.float32),
        in_specs=[
            pl.BlockSpec(memory_space=pltpu.VMEM),
            pl.BlockSpec(memory_space=pltpu.VMEM),
            pl.BlockSpec(memory_space=pltpu.SMEM),
            pl.BlockSpec(memory_space=pltpu.SMEM),
        ],
        out_specs=pl.BlockSpec(memory_space=pltpu.VMEM),
        scratch_shapes=[
            pltpu.VMEM((2, mb, n), COMM_DTYPE),
            pltpu.SemaphoreType.DMA((2,)),
            pltpu.SemaphoreType.DMA((2,)),
            pltpu.SemaphoreType.REGULAR,
        ],
        compiler_params=pltpu.CompilerParams(collective_id=0),
    )(x, w_mat, scale_x, scale_w)


# device time: 431597 ns/iter; 1.7705x vs baseline; 1.7705x over previous
import functools

import jax
import jax.numpy as jnp
from jax import lax
from jax.experimental import pallas as pl
from jax.experimental.pallas import tpu as pltpu

N_DEV = 16
COMM_DTYPE = jnp.bfloat16


def kernel(x, w_mat, scale_x, scale_w):
    m, k_local = x.shape
    _, n = w_mat.shape
    mb = m // N_DEV
    nh = n // 2

    def body(x_ref, w_ref, sx_ref, sw_ref, out_ref,
             comm_f, comm_b, send_sems_f, recv_sems_f, send_sems_b,
             recv_sems_b, credit_f, credit_b):
        my = lax.axis_index("i")
        left = lax.rem(my - 1 + N_DEV, N_DEV)
        right = lax.rem(my + 1, N_DEV)

        barrier_sem = pltpu.get_barrier_semaphore()
        for nbr in (left, right):
            pl.semaphore_signal(barrier_sem, inc=1, device_id=(nbr,),
                                device_id_type=pl.DeviceIdType.MESH)
        pl.semaphore_wait(barrier_sem, 2)

        w_bf16 = w_ref[...].astype(jnp.bfloat16)

        def contrib(o, lo):
            xs = x_ref[pl.ds(o * mb, mb), :].astype(jnp.bfloat16)
            return lax.dot_general(
                xs, w_bf16[:, lo:lo + nh], (((1,), (0,)), ((), ())),
                preferred_element_type=jnp.float32)

        of0 = lax.rem(my - 1 + N_DEV, N_DEV)
        ob0 = lax.rem(my + 1, N_DEV)
        comm_f[1] = contrib(of0, 0).astype(COMM_DTYPE)
        comm_b[1] = contrib(ob0, nh).astype(COMM_DTYPE)

        for s in range(N_DEV - 1):
            src_slot = (s - 1) % 2
            dst_slot = s % 2
            if s >= 2:
                pl.semaphore_wait(credit_f, 1)
                pl.semaphore_wait(credit_b, 1)
            rdma_f = pltpu.make_async_remote_copy(
                src_ref=comm_f.at[src_slot],
                dst_ref=comm_f.at[dst_slot],
                send_sem=send_sems_f.at[src_slot],
                recv_sem=recv_sems_f.at[dst_slot],
                device_id=(right,),
                device_id_type=pl.DeviceIdType.MESH,
            )
            rdma_b = pltpu.make_async_remote_copy(
                src_ref=comm_b.at[src_slot],
                dst_ref=comm_b.at[dst_slot],
                send_sem=send_sems_b.at[src_slot],
                recv_sem=recv_sems_b.at[dst_slot],
                device_id=(left,),
                device_id_type=pl.DeviceIdType.MESH,
            )
            rdma_f.start()
            rdma_b.start()

            of = lax.rem(my - 2 - s + 2 * N_DEV, N_DEV)
            ob = lax.rem(my + 2 + s, N_DEV)

            rdma_f.wait()
            if 1 <= s <= 13:
                pl.semaphore_signal(credit_f, inc=1, device_id=(left,),
                                    device_id_type=pl.DeviceIdType.MESH)
            merged_f = comm_f[dst_slot].astype(jnp.float32) + contrib(of, 0)
            if s < N_DEV - 2:
                comm_f[dst_slot] = merged_f.astype(COMM_DTYPE)

            rdma_b.wait()
            if 1 <= s <= 13:
                pl.semaphore_signal(credit_b, inc=1, device_id=(right,),
                                    device_id_type=pl.DeviceIdType.MESH)
            merged_b = comm_b[dst_slot].astype(jnp.float32) + contrib(ob, nh)
            if s < N_DEV - 2:
                comm_b[dst_slot] = merged_b.astype(COMM_DTYPE)
            else:
                scale = sx_ref[0] * sw_ref[0]
                y_f = merged_f * scale
                y_b = merged_b * scale
                out_ref[:, 0:nh] = y_f * jax.nn.sigmoid(y_f)
                out_ref[:, nh:n] = y_b * jax.nn.sigmoid(y_b)

        @functools.partial(pl.run_scoped,
                           second_barrier=pltpu.SemaphoreType.REGULAR)
        def _(second_barrier):
            for nbr in (left, right):
                pl.semaphore_signal(second_barrier, inc=1, device_id=(nbr,),
                                    device_id_type=pl.DeviceIdType.MESH)
            pl.semaphore_wait(second_barrier, 2)

    return pl.pallas_call(
        body,
        out_shape=jax.ShapeDtypeStruct((mb, n), jnp.float32),
        in_specs=[
            pl.BlockSpec(memory_space=pltpu.VMEM),
            pl.BlockSpec(memory_space=pltpu.VMEM),
            pl.BlockSpec(memory_space=pltpu.SMEM),
            pl.BlockSpec(memory_space=pltpu.SMEM),
        ],
        out_specs=pl.BlockSpec(memory_space=pltpu.VMEM),
        scratch_shapes=[
            pltpu.VMEM((2, mb, nh), COMM_DTYPE),
            pltpu.VMEM((2, mb, nh), COMM_DTYPE),
            pltpu.SemaphoreType.DMA((2,)),
            pltpu.SemaphoreType.DMA((2,)),
            pltpu.SemaphoreType.DMA((2,)),
            pltpu.SemaphoreType.DMA((2,)),
            pltpu.SemaphoreType.REGULAR,
            pltpu.SemaphoreType.REGULAR,
        ],
        compiler_params=pltpu.CompilerParams(collective_id=0),
    )(x, w_mat, scale_x, scale_w)


# device time: 359137 ns/iter; 2.1278x vs baseline; 1.2018x over previous
import functools

import jax
import jax.numpy as jnp
from jax import lax
from jax.experimental import pallas as pl
from jax.experimental.pallas import tpu as pltpu

N_DEV = 16
N_STREAM = 4
COMM_DTYPE = jnp.bfloat16


def kernel(x, w_mat, scale_x, scale_w):
    m, k_local = x.shape
    _, n = w_mat.shape
    mb = m // N_DEV
    nq = n // N_STREAM

    def body(x_ref, w_ref, sx_ref, sw_ref, out_ref, *scratch):
        comms = scratch[0:4]
        send_sems = scratch[4:8]
        recv_sems = scratch[8:12]
        credits = scratch[12:16]

        my = lax.axis_index("i")
        left = lax.rem(my - 1 + N_DEV, N_DEV)
        right = lax.rem(my + 1, N_DEV)

        fwd = (True, True, False, False)
        dst_dev = tuple(right if f else left for f in fwd)
        up_dev = tuple(left if f else right for f in fwd)

        barrier_sem = pltpu.get_barrier_semaphore()
        for nbr in (left, right):
            pl.semaphore_signal(barrier_sem, inc=1, device_id=(nbr,),
                                device_id_type=pl.DeviceIdType.MESH)
        pl.semaphore_wait(barrier_sem, 2)

        w_bf16 = w_ref[...].astype(jnp.bfloat16)

        def contrib(o, k):
            xs = x_ref[pl.ds(o * mb, mb), :].astype(jnp.bfloat16)
            return lax.dot_general(
                xs, w_bf16[:, k * nq:(k + 1) * nq], (((1,), (0,)), ((), ())),
                preferred_element_type=jnp.float32)

        def chunk_at(s, k):
            if fwd[k]:
                return lax.rem(my - 2 - s + 2 * N_DEV, N_DEV)
            return lax.rem(my + 2 + s, N_DEV)

        def make_rdma(s, k):
            return pltpu.make_async_remote_copy(
                src_ref=comms[k].at[(s - 1) % 2],
                dst_ref=comms[k].at[s % 2],
                send_sem=send_sems[k].at[(s - 1) % 2],
                recv_sem=recv_sems[k].at[s % 2],
                device_id=(dst_dev[k],),
                device_id_type=pl.DeviceIdType.MESH,
            )

        for k in range(N_STREAM):
            comms[k][1] = contrib(chunk_at(-1, k), k).astype(COMM_DTYPE)
        for k in range(N_STREAM):
            make_rdma(0, k).start()

        scale = sx_ref[0] * sw_ref[0]

        order = (0, 2, 1, 3)
        for s in range(N_DEV - 1):
            for k in order:
                make_rdma(s, k).wait()
                if 1 <= s <= 13:
                    pl.semaphore_signal(
                        credits[k], inc=1, device_id=(up_dev[k],),
                        device_id_type=pl.DeviceIdType.MESH)
                merged = (comms[k][s % 2].astype(jnp.float32)
                          + contrib(chunk_at(s, k), k))
                if s < N_DEV - 2:
                    comms[k][s % 2] = merged.astype(COMM_DTYPE)
                    if s + 1 >= 2:
                        pl.semaphore_wait(credits[k], 1)
                    make_rdma(s + 1, k).start()
                else:
                    y = merged * scale
                    out_ref[:, k * nq:(k + 1) * nq] = y * jax.nn.sigmoid(y)

        @functools.partial(pl.run_scoped,
                           second_barrier=pltpu.SemaphoreType.REGULAR)
        def _(second_barrier):
            for nbr in (left, right):
                pl.semaphore_signal(second_barrier, inc=1, device_id=(nbr,),
                                    device_id_type=pl.DeviceIdType.MESH)
            pl.semaphore_wait(second_barrier, 2)

    return pl.pallas_call(
        body,
        out_shape=jax.ShapeDtypeStruct((mb, n), jnp.float32),
        in_specs=[
            pl.BlockSpec(memory_space=pltpu.VMEM),
            pl.BlockSpec(memory_space=pltpu.VMEM),
            pl.BlockSpec(memory_space=pltpu.SMEM),
            pl.BlockSpec(memory_space=pltpu.SMEM),
        ],
        out_specs=pl.BlockSpec(memory_space=pltpu.VMEM),
        scratch_shapes=(
            [pltpu.VMEM((2, mb, nq), COMM_DTYPE) for _ in range(N_STREAM)]
            + [pltpu.SemaphoreType.DMA((2,)) for _ in range(N_STREAM)]
            + [pltpu.SemaphoreType.DMA((2,)) for _ in range(N_STREAM)]
            + [pltpu.SemaphoreType.REGULAR for _ in range(N_STREAM)]
        ),
        compiler_params=pltpu.CompilerParams(collective_id=0),
    )(x, w_mat, scale_x, scale_w)


# device time: 359123 ns/iter; 2.1278x vs baseline; 1.0000x over previous
import functools

import jax
import jax.numpy as jnp
from jax import lax
from jax.experimental import pallas as pl
from jax.experimental.pallas import tpu as pltpu

N_DEV = 16
N_STREAM = 4
COMM_DTYPE = jnp.bfloat16


def kernel(x, w_mat, scale_x, scale_w):
    m, k_local = x.shape
    _, n = w_mat.shape
    mb = m // N_DEV
    nq = n // N_STREAM

    def body(x_ref, w_ref, sx_ref, sw_ref, out_ref, *scratch):
        comms = scratch[0:4]
        send_sems = scratch[4:8]
        recv_sems = scratch[8:12]
        credits = scratch[12:16]

        my = lax.axis_index("i")
        left = lax.rem(my - 1 + N_DEV, N_DEV)
        right = lax.rem(my + 1, N_DEV)

        fwd = (True, True, False, False)
        dst_dev = tuple(right if f else left for f in fwd)
        up_dev = tuple(left if f else right for f in fwd)

        def contrib(o, k):
            xs = x_ref[pl.ds(o * mb, mb), :]
            ws = w_ref[:, k * nq:(k + 1) * nq]
            return lax.dot_general(
                xs, ws, (((1,), (0,)), ((), ())),
                preferred_element_type=jnp.float32)

        def chunk_at(s, k):
            if fwd[k]:
                return lax.rem(my - 2 - s + 2 * N_DEV, N_DEV)
            return lax.rem(my + 2 + s, N_DEV)

        def make_rdma(s, k):
            return pltpu.make_async_remote_copy(
                src_ref=comms[k].at[(s - 1) % 2],
                dst_ref=comms[k].at[s % 2],
                send_sem=send_sems[k].at[(s - 1) % 2],
                recv_sem=recv_sems[k].at[s % 2],
                device_id=(dst_dev[k],),
                device_id_type=pl.DeviceIdType.MESH,
            )

        for k in range(N_STREAM):
            comms[k][1] = contrib(chunk_at(-1, k), k).astype(COMM_DTYPE)

        barrier_sem = pltpu.get_barrier_semaphore()
        for nbr in (left, right):
            pl.semaphore_signal(barrier_sem, inc=1, device_id=(nbr,),
                                device_id_type=pl.DeviceIdType.MESH)
        pl.semaphore_wait(barrier_sem, 2)

        for k in range(N_STREAM):
            make_rdma(0, k).start()

        scale = sx_ref[0] * sw_ref[0]

        order = (0, 2, 1, 3)
        for s in range(N_DEV - 1):
            for k in order:
                make_rdma(s, k).wait()
                if 1 <= s <= 13:
                    pl.semaphore_signal(
                        credits[k], inc=1, device_id=(up_dev[k],),
                        device_id_type=pl.DeviceIdType.MESH)
                merged = (comms[k][s % 2].astype(jnp.float32)
                          + contrib(chunk_at(s, k), k))
                if s < N_DEV - 2:
                    comms[k][s % 2] = merged.astype(COMM_DTYPE)
                    if s + 1 >= 2:
                        pl.semaphore_wait(credits[k], 1)
                    make_rdma(s + 1, k).start()
                else:
                    y = merged * scale
                    out_ref[:, k * nq:(k + 1) * nq] = y * jax.nn.sigmoid(y)

        @functools.partial(pl.run_scoped,
                           second_barrier=pltpu.SemaphoreType.REGULAR)
        def _(second_barrier):
            for nbr in (left, right):
                pl.semaphore_signal(second_barrier, inc=1, device_id=(nbr,),
                                    device_id_type=pl.DeviceIdType.MESH)
            pl.semaphore_wait(second_barrier, 2)

    return pl.pallas_call(
        body,
        out_shape=jax.ShapeDtypeStruct((mb, n), jnp.float32),
        in_specs=[
            pl.BlockSpec(memory_space=pltpu.VMEM),
            pl.BlockSpec(memory_space=pltpu.VMEM),
            pl.BlockSpec(memory_space=pltpu.SMEM),
            pl.BlockSpec(memory_space=pltpu.SMEM),
        ],
        out_specs=pl.BlockSpec(memory_space=pltpu.VMEM),
        scratch_shapes=(
            [pltpu.VMEM((2, mb, nq), COMM_DTYPE) for _ in range(N_STREAM)]
            + [pltpu.SemaphoreType.DMA((2,)) for _ in range(N_STREAM)]
            + [pltpu.SemaphoreType.DMA((2,)) for _ in range(N_STREAM)]
            + [pltpu.SemaphoreType.REGULAR for _ in range(N_STREAM)]
        ),
        compiler_params=pltpu.CompilerParams(collective_id=0),
    )(x, w_mat, scale_x, scale_w)
